# baseline (speedup 1.0000x reference)
import functools

import jax
import jax.numpy as jnp
from jax import lax
from jax.experimental import pallas as pl
from jax.experimental.pallas import tpu as pltpu
from jax.experimental.pallas import tpu_sc as plsc

B, R, N, DIM, E = 1024, 8, 8, 64, 100000
DP = 128

NC, NS = 2, 16
NW = NC * NS
N0 = (B * R) // NW
N1 = (B * R * N) // NW
N2 = (B * R * N * N) // NW
CH = 256
NG = CH // N


def _sc_gather(table, i0, i1, i2):
    mesh = plsc.VectorSubcoreMesh(core_axis_name="c", subcore_axis_name="s")

    @functools.partial(
        pl.kernel,
        out_type=(
            jax.ShapeDtypeStruct((B * R, DP), jnp.float32),
            jax.ShapeDtypeStruct((B * R * N, DP), jnp.float32),
            jax.ShapeDtypeStruct((B * R * N, DP), jnp.float32),
        ),
        mesh=mesh,
        scratch_types=[
            pltpu.VMEM((N2,), jnp.int32),
            pltpu.VMEM((CH, DP), jnp.float32),
            pltpu.VMEM((CH, DP), jnp.float32),
            pltpu.VMEM((NG, DP), jnp.float32),
            pltpu.VMEM((NG, DP), jnp.float32),
            pltpu.SemaphoreType.DMA,
            pltpu.SemaphoreType.DMA,
            pltpu.SemaphoreType.DMA,
            pltpu.SemaphoreType.DMA,
        ],
        compiler_params=pltpu.CompilerParams(
            use_tc_tiling_on_sc=True, needs_layout_passes=False),
    )
    def k(table_h, i0_h, i1_h, i2_h, e0_h, e1_h, s2_h,
          idx_v, rows_a, rows_b, acc_a, acc_b, gsem_a, gsem_b, wsem_a, wsem_b):
        wid = lax.axis_index("s") * NC + lax.axis_index("c")
        rows = (rows_a, rows_b)
        accs = (acc_a, acc_b)
        gsems = (gsem_a, gsem_b)
        wsems = (wsem_a, wsem_b)

        pltpu.sync_copy(i0_h.at[pl.ds(pl.multiple_of(wid * N0, N0), N0)],
                        idx_v.at[pl.ds(0, N0)])
        pltpu.async_copy(table_h.at[idx_v.at[pl.ds(0, N0)]],
                         rows_a, gsem_a).wait()
        w0a = pltpu.async_copy(
            rows_a, e0_h.at[pl.ds(pl.multiple_of(wid * N0, N0), N0)], wsem_a)

        pltpu.sync_copy(i1_h.at[pl.ds(pl.multiple_of(wid * N1, N1), N1)],
                        idx_v.at[pl.ds(0, N1)])
        w0a.wait()
        nch1 = N1 // CH
        g_h = [None] * nch1
        wr_h = [None] * nch1
        g_h[0] = pltpu.async_copy(
            table_h.at[idx_v.at[pl.ds(0, CH)]], rows[0], gsems[0])
        for c in range(nch1):
            g_h[c].wait()
            if c + 1 < nch1:
                if c >= 1:
                    wr_h[c - 1].wait()
                g_h[c + 1] = pltpu.async_copy(
                    table_h.at[idx_v.at[pl.ds((c + 1) * CH, CH)]],
                    rows[(c + 1) % 2], gsems[(c + 1) % 2])
            wr_h[c] = pltpu.async_copy(
                rows[c % 2],
                e1_h.at[pl.ds(pl.multiple_of(wid * N1 + c * CH, CH), CH)],
                wsems[c % 2])
        wr_h[nch1 - 2].wait()
        wr_h[nch1 - 1].wait()

        pltpu.sync_copy(i2_h.at[pl.ds(pl.multiple_of(wid * N2, N2), N2)],
                        idx_v)
        nch2 = N2 // CH
        gbase = (wid * N2) // N
        pltpu.async_copy(
            acc_a, s2_h.at[pl.ds(pl.multiple_of(gbase, NG), NG)], wsem_a)
        pltpu.async_copy(
            acc_b, s2_h.at[pl.ds(pl.multiple_of(gbase + NG, NG), NG)],
            wsem_b)
        pltpu.async_copy(table_h.at[idx_v.at[pl.ds(0, CH)]],
                         rows[0], gsems[0])
        pltpu.async_copy(table_h.at[idx_v.at[pl.ds(CH, CH)]],
                         rows[1], gsems[1])

        def half(t, bf, par):
            pltpu.make_async_copy(
                table_h.at[pl.ds(0, CH)], rows[bf], gsems[bf]).wait()
            pltpu.make_async_copy(
                accs[bf], s2_h.at[pl.ds(0, NG)], wsems[bf]).wait()

            def grp(gt, carry):
                rslab = rows[bf].at[pl.ds(pl.multiple_of(gt * 64, 64), 64)]
                aslab = accs[bf].at[pl.ds(pl.multiple_of(gt * 8, 8), 8)]
                for j in range(8):
                    for lc in range(DIM // 16):
                        acc = rslab[j * N, pl.ds(lc * 16, 16)]
                        for kk in range(1, N):
                            acc = acc + rslab[j * N + kk, pl.ds(lc * 16, 16)]
                        aslab[j, pl.ds(lc * 16, 16)] = acc
                return carry

            lax.fori_loop(0, NG // 8, grp, 0)
            pltpu.async_copy(
                accs[bf],
                s2_h.at[pl.ds(
                    pl.multiple_of(gbase + (2 * t + par) * NG, NG), NG)],
                wsems[bf])
            nxt = jnp.minimum((2 * t + par + 2) * CH, N2 - CH)
            pltpu.async_copy(
                table_h.at[idx_v.at[pl.ds(pl.multiple_of(nxt, CH), CH)]],
                rows[bf], gsems[bf])

        def pair(t, carry):
            half(t, 0, 0)
            half(t, 1, 1)
            return carry

        lax.fori_loop(0, nch2 // 2, pair, 0)
        for bf in range(2):
            pltpu.make_async_copy(
                table_h.at[pl.ds(0, CH)], rows[bf], gsems[bf]).wait()
            pltpu.make_async_copy(
                accs[bf], s2_h.at[pl.ds(0, NG)], wsems[bf]).wait()

    return k(table, i0, i1, i2)


NB = 256


def _tc_compute(e0, e1, s2, Wm, bias_row, rule_rows):
    NBR = NB * R
    NBL = NB * R * N

    def body(e0_ref, e1_ref, s2_ref, w_ref, b_ref, rw_ref, out_ref):
        inv = jnp.float32(1.0 / N)
        Wf = w_ref[...].astype(jnp.bfloat16)
        bb = b_ref[...]
        e1v = e1_ref[:, :DIM]
        x1 = jnp.concatenate([e1v, s2_ref[:, :DIM] * inv], axis=-1)
        h1 = jax.nn.relu(
            lax.dot(x1.astype(jnp.bfloat16), Wf,
                    preferred_element_type=jnp.float32) + bb)
        m1 = h1.reshape(NBR, N, DIM).sum(axis=1) * inv
        m0 = e1v.reshape(NBR, N, DIM).sum(axis=1) * inv
        x0 = jnp.concatenate([e0_ref[:, :DIM], m0], axis=-1)
        h0 = jax.nn.relu(
            lax.dot(x0.astype(jnp.bfloat16), Wf,
                    preferred_element_type=jnp.float32) + bb)
        xo = jnp.concatenate([h0, m1], axis=-1)
        o = jnp.tanh(
            lax.dot(xo.astype(jnp.bfloat16), Wf,
                    preferred_element_type=jnp.float32) + bb)
        res = (o.reshape(NB, R, DIM) * rw_ref[...].reshape(1, R, DIM))
        out_ref[...] = res.sum(axis=1)

    return pl.pallas_call(
        body,
        grid=(B // NB,),
        in_specs=[
            pl.BlockSpec((NBR, DP), lambda i: (i, 0)),
            pl.BlockSpec((NBL, DP), lambda i: (i, 0)),
            pl.BlockSpec((NBL, DP), lambda i: (i, 0)),
            pl.BlockSpec((2 * DIM, DIM), lambda i: (0, 0)),
            pl.BlockSpec((1, DIM), lambda i: (0, 0)),
            pl.BlockSpec((R, DIM), lambda i: (0, 0)),
        ],
        out_specs=pl.BlockSpec((NB, DIM), lambda i: (i, 0)),
        out_shape=jax.ShapeDtypeStruct((B, DIM), jnp.float32),
    )(e0, e1, s2, Wm, bias_row, rule_rows)


def kernel(idx0, idx1, idx2, ent_embed, rule_w, W, b):
    tbl = jnp.pad(ent_embed, ((0, 0), (0, DP - DIM)))
    i0 = idx0.astype(jnp.int32).reshape(-1)
    i1 = idx1.astype(jnp.int32).reshape(-1)
    i2 = idx2.astype(jnp.int32).reshape(-1)
    e0, e1, s2 = _sc_gather(tbl, i0, i1, i2)
    bias_row = b.reshape(1, DIM)
    rule_rows = jnp.broadcast_to(rule_w.reshape(R, 1), (R, DIM))
    return _tc_compute(e0, e1, s2, W, bias_row, rule_rows)


# speedup vs baseline: 1.0132x; 1.0132x over previous
import functools

import jax
import jax.numpy as jnp
from jax import lax
from jax.experimental import pallas as pl
from jax.experimental.pallas import tpu as pltpu
from jax.experimental.pallas import tpu_sc as plsc

B, R, N, DIM, E = 1024, 8, 8, 64, 100000
DP = 128

NC, NS = 2, 16
NW = NC * NS
N0 = (B * R) // NW
N1 = (B * R * N) // NW
N2 = (B * R * N * N) // NW
CH = 256
NG = CH // N


def _sc_gather(table, i0, i1, i2):
    mesh = plsc.VectorSubcoreMesh(core_axis_name="c", subcore_axis_name="s")

    @functools.partial(
        pl.kernel,
        out_type=(
            jax.ShapeDtypeStruct((B * R, DP), jnp.float32),
            jax.ShapeDtypeStruct((B * R * N, DP), jnp.float32),
            jax.ShapeDtypeStruct((B * R * N, DP), jnp.float32),
        ),
        mesh=mesh,
        scratch_types=[
            pltpu.VMEM((N2,), jnp.int32),
            pltpu.VMEM((CH, DP), jnp.float32),
            pltpu.VMEM((CH, DP), jnp.float32),
            pltpu.VMEM((NG, DP), jnp.float32),
            pltpu.VMEM((NG, DP), jnp.float32),
            pltpu.SemaphoreType.DMA,
            pltpu.SemaphoreType.DMA,
            pltpu.SemaphoreType.DMA,
            pltpu.SemaphoreType.DMA,
        ],
        compiler_params=pltpu.CompilerParams(
            use_tc_tiling_on_sc=True, needs_layout_passes=False),
    )
    def k(table_h, i0_h, i1_h, i2_h, e0_h, e1_h, s2_h,
          idx_v, rows_a, rows_b, acc_a, acc_b, gsem_a, gsem_b, wsem_a, wsem_b):
        wid = lax.axis_index("s") * NC + lax.axis_index("c")
        rows = (rows_a, rows_b)
        accs = (acc_a, acc_b)
        gsems = (gsem_a, gsem_b)
        wsems = (wsem_a, wsem_b)

        pltpu.sync_copy(i0_h.at[pl.ds(pl.multiple_of(wid * N0, N0), N0)],
                        idx_v.at[pl.ds(0, N0)])
        pltpu.async_copy(table_h.at[idx_v.at[pl.ds(0, N0)]],
                         rows_a, gsem_a).wait()
        w0a = pltpu.async_copy(
            rows_a, e0_h.at[pl.ds(pl.multiple_of(wid * N0, N0), N0)], wsem_a)

        pltpu.sync_copy(i1_h.at[pl.ds(pl.multiple_of(wid * N1, N1), N1)],
                        idx_v.at[pl.ds(0, N1)])
        w0a.wait()
        nch1 = N1 // CH
        g_h = [None] * nch1
        wr_h = [None] * nch1
        g_h[0] = pltpu.async_copy(
            table_h.at[idx_v.at[pl.ds(0, CH)]], rows[0], gsems[0])
        for c in range(nch1):
            g_h[c].wait()
            if c + 1 < nch1:
                if c >= 1:
                    wr_h[c - 1].wait()
                g_h[c + 1] = pltpu.async_copy(
                    table_h.at[idx_v.at[pl.ds((c + 1) * CH, CH)]],
                    rows[(c + 1) % 2], gsems[(c + 1) % 2])
            wr_h[c] = pltpu.async_copy(
                rows[c % 2],
                e1_h.at[pl.ds(pl.multiple_of(wid * N1 + c * CH, CH), CH)],
                wsems[c % 2])
        wr_h[nch1 - 2].wait()
        wr_h[nch1 - 1].wait()

        pltpu.sync_copy(i2_h.at[pl.ds(pl.multiple_of(wid * N2, N2), N2)],
                        idx_v)
        nch2 = N2 // CH
        gbase = (wid * N2) // N
        pltpu.async_copy(
            acc_a, s2_h.at[pl.ds(pl.multiple_of(gbase, NG), NG)], wsem_a)
        pltpu.async_copy(
            acc_b, s2_h.at[pl.ds(pl.multiple_of(gbase + NG, NG), NG)],
            wsem_b)
        pltpu.async_copy(table_h.at[idx_v.at[pl.ds(0, CH)]],
                         rows[0], gsems[0])
        pltpu.async_copy(table_h.at[idx_v.at[pl.ds(CH, CH)]],
                         rows[1], gsems[1])

        def half(t, bf, par):
            pltpu.make_async_copy(
                table_h.at[pl.ds(0, CH)], rows[bf], gsems[bf]).wait()
            pltpu.make_async_copy(
                accs[bf], s2_h.at[pl.ds(0, NG)], wsems[bf]).wait()

            def grp(gt, carry):
                rslab = rows[bf].at[pl.ds(pl.multiple_of(gt * 64, 64), 64)]
                aslab = accs[bf].at[pl.ds(pl.multiple_of(gt * 8, 8), 8)]
                for j in range(8):
                    for lc in range(DIM // 16):
                        acc = rslab[j * N, pl.ds(lc * 16, 16)]
                        for kk in range(1, N):
                            acc = acc + rslab[j * N + kk, pl.ds(lc * 16, 16)]
                        aslab[j, pl.ds(lc * 16, 16)] = acc
                return carry

            lax.fori_loop(0, NG // 8, grp, 0)
            pltpu.async_copy(
                accs[bf],
                s2_h.at[pl.ds(
                    pl.multiple_of(gbase + (2 * t + par) * NG, NG), NG)],
                wsems[bf])
            nxt = jnp.minimum((2 * t + par + 2) * CH, N2 - CH)
            pltpu.async_copy(
                table_h.at[idx_v.at[pl.ds(pl.multiple_of(nxt, CH), CH)]],
                rows[bf], gsems[bf])

        def pair(t, carry):
            half(t, 0, 0)
            half(t, 1, 1)
            return carry

        lax.fori_loop(0, nch2 // 2, pair, 0)
        for bf in range(2):
            pltpu.make_async_copy(
                table_h.at[pl.ds(0, CH)], rows[bf], gsems[bf]).wait()
            pltpu.make_async_copy(
                accs[bf], s2_h.at[pl.ds(0, NG)], wsems[bf]).wait()

    return k(table, i0, i1, i2)


NB = 128


def _tc_compute(e0, e1, s2, Wm, bias_row, rule_rows):
    NBR = NB * R
    NBL = NB * R * N

    def body(e0_ref, e1_ref, s2_ref, w_ref, b_ref, rw_ref, out_ref):
        inv = jnp.float32(1.0 / N)
        Wf = w_ref[...].astype(jnp.bfloat16)
        bb = b_ref[...]
        e1v = e1_ref[:, :DIM]
        x1 = jnp.concatenate([e1v, s2_ref[:, :DIM] * inv], axis=-1)
        h1 = jax.nn.relu(
            lax.dot(x1.astype(jnp.bfloat16), Wf,
                    preferred_element_type=jnp.float32) + bb)
        m1 = h1.reshape(NBR, N, DIM).sum(axis=1) * inv
        m0 = e1v.reshape(NBR, N, DIM).sum(axis=1) * inv
        x0 = jnp.concatenate([e0_ref[:, :DIM], m0], axis=-1)
        h0 = jax.nn.relu(
            lax.dot(x0.astype(jnp.bfloat16), Wf,
                    preferred_element_type=jnp.float32) + bb)
        xo = jnp.concatenate([h0, m1], axis=-1)
        o = jnp.tanh(
            lax.dot(xo.astype(jnp.bfloat16), Wf,
                    preferred_element_type=jnp.float32) + bb)
        res = (o.reshape(NB, R, DIM) * rw_ref[...].reshape(1, R, DIM))
        out_ref[...] = res.sum(axis=1)

    return pl.pallas_call(
        body,
        grid=(B // NB,),
        in_specs=[
            pl.BlockSpec((NBR, DP), lambda i: (i, 0)),
            pl.BlockSpec((NBL, DP), lambda i: (i, 0)),
            pl.BlockSpec((NBL, DP), lambda i: (i, 0)),
            pl.BlockSpec((2 * DIM, DIM), lambda i: (0, 0)),
            pl.BlockSpec((1, DIM), lambda i: (0, 0)),
            pl.BlockSpec((R, DIM), lambda i: (0, 0)),
        ],
        out_specs=pl.BlockSpec((NB, DIM), lambda i: (i, 0)),
        out_shape=jax.ShapeDtypeStruct((B, DIM), jnp.float32),
    )(e0, e1, s2, Wm, bias_row, rule_rows)


def kernel(idx0, idx1, idx2, ent_embed, rule_w, W, b):
    tbl = jnp.pad(ent_embed, ((0, 0), (0, DP - DIM)))
    i0 = idx0.astype(jnp.int32).reshape(-1)
    i1 = idx1.astype(jnp.int32).reshape(-1)
    i2 = idx2.astype(jnp.int32).reshape(-1)
    e0, e1, s2 = _sc_gather(tbl, i0, i1, i2)
    bias_row = b.reshape(1, DIM)
    rule_rows = jnp.broadcast_to(rule_w.reshape(R, 1), (R, DIM))
    return _tc_compute(e0, e1, s2, W, bias_row, rule_rows)
